# baseline (speedup 1.0000x reference)
import functools

import jax
import jax.numpy as jnp
from jax import lax
from jax.experimental import pallas as pl
from jax.experimental.pallas import tpu as pltpu
from jax.experimental.pallas import tpu_sc as plsc

_EMBED_DIM = 64
_BATCH = 4096
_HIST = 200

_NC = 2
_NS = 16
_NW = _NC * _NS
_ROWS_W = _BATCH // _NW
_NBUF = 2
_NGROUP = _ROWS_W // _NBUF


def _make_kernel():
  mesh = plsc.VectorSubcoreMesh(core_axis_name="c", subcore_axis_name="s")

  @functools.partial(
      pl.kernel,
      mesh=mesh,
      compiler_params=pltpu.CompilerParams(use_tc_tiling_on_sc=False),
      out_type=jax.ShapeDtypeStruct((_BATCH, _HIST, _EMBED_DIM), jnp.float32),
      scratch_types=[
          pltpu.VMEM((_ROWS_W, _HIST), jnp.int32),
          pltpu.VMEM((2, _NBUF, _HIST, _EMBED_DIM), jnp.float32),
          pltpu.SemaphoreType.DMA((2, _NBUF)),
          pltpu.SemaphoreType.DMA((2, _NBUF)),
      ],
  )
  def k(idx_hbm, table_hbm, out_hbm, idx_v, rows_v, gsem, ssem):
    wid = lax.axis_index("s") * _NC + lax.axis_index("c")
    base = wid * _ROWS_W

    pltpu.sync_copy(idx_hbm.at[pl.ds(base, _ROWS_W)], idx_v)

    def fire_gather(row, p, b):
      pltpu.async_copy(
          table_hbm.at[idx_v.at[row]], rows_v.at[p, b], gsem.at[p, b])

    def fire_store(row, p, b):
      pltpu.async_copy(
          rows_v.at[p, b], out_hbm.at[base + row], ssem.at[p, b])

    def wait_gather(p, b):
      pltpu.make_async_copy(
          table_hbm.at[idx_v.at[0]], rows_v.at[p, b], gsem.at[p, b]).wait()

    def wait_store(p, b):
      pltpu.make_async_copy(
          rows_v.at[p, b], out_hbm.at[base], ssem.at[p, b]).wait()

    for b in range(_NBUF):
      fire_gather(b, 0, b)

    def body(j, carry):
      for p in (0, 1):
        i = 2 * j + p
        q = 1 - p
        for b in range(_NBUF):
          @pl.when(i > 0)
          def _():
            wait_store(q, b)

          @pl.when(i < _NGROUP - 1)
          def _():
            fire_gather((i + 1) * _NBUF + b, q, b)

          wait_gather(p, b)
          fire_store(i * _NBUF + b, p, b)
      return carry

    lax.fori_loop(0, _NGROUP // 2, body, 0)

    for b in range(_NBUF):
      wait_store(1, b)

  return k


_kernel_call = _make_kernel()


def kernel(idx, weight):
  return _kernel_call(idx.astype(jnp.int32), weight)


# speedup vs baseline: 1.3281x; 1.3281x over previous
import functools

import jax
import jax.numpy as jnp
from jax import lax
from jax.experimental import pallas as pl
from jax.experimental.pallas import tpu as pltpu
from jax.experimental.pallas import tpu_sc as plsc

_EMBED_DIM = 64
_BATCH = 4096
_HIST = 200

_NC = 2
_NS = 16
_NW = _NC * _NS
_ROWS_W = _BATCH // _NW
_NBUF = 2
_NGROUP = _ROWS_W // _NBUF


def _make_kernel():
  mesh = plsc.VectorSubcoreMesh(core_axis_name="c", subcore_axis_name="s")

  @functools.partial(
      pl.kernel,
      mesh=mesh,
      compiler_params=pltpu.CompilerParams(use_tc_tiling_on_sc=False),
      out_type=jax.ShapeDtypeStruct((_BATCH, _HIST, 2 * _EMBED_DIM),
                                    jnp.float32),
      scratch_types=[
          pltpu.VMEM((_ROWS_W, _HIST), jnp.int32),
          pltpu.VMEM((2, _NBUF, _HIST, _EMBED_DIM), jnp.float32),
          pltpu.SemaphoreType.DMA((2, _NBUF)),
          pltpu.SemaphoreType.DMA((2, _NBUF)),
      ],
  )
  def k(idx_hbm, table_hbm, out_hbm, idx_v, rows_v, gsem, ssem):
    wid = lax.axis_index("s") * _NC + lax.axis_index("c")
    base = wid * _ROWS_W

    pltpu.sync_copy(idx_hbm.at[pl.ds(base, _ROWS_W)], idx_v)

    def fire_gather(row, p, b):
      pltpu.async_copy(
          table_hbm.at[idx_v.at[row]], rows_v.at[p, b], gsem.at[p, b])

    def fire_store(row, p, b):
      pltpu.async_copy(
          rows_v.at[p, b],
          out_hbm.at[base + row, :, pl.ds(0, _EMBED_DIM)],
          ssem.at[p, b])

    def wait_gather(p, b):
      pltpu.make_async_copy(
          table_hbm.at[idx_v.at[0]], rows_v.at[p, b], gsem.at[p, b]).wait()

    def wait_store(p, b):
      pltpu.make_async_copy(
          rows_v.at[p, b],
          out_hbm.at[base, :, pl.ds(0, _EMBED_DIM)],
          ssem.at[p, b]).wait()

    for b in range(_NBUF):
      fire_gather(b, 0, b)

    def body(j, carry):
      for p in (0, 1):
        i = 2 * j + p
        q = 1 - p
        for b in range(_NBUF):
          @pl.when(i > 0)
          def _():
            wait_store(q, b)

          @pl.when(i < _NGROUP - 1)
          def _():
            fire_gather((i + 1) * _NBUF + b, q, b)

          wait_gather(p, b)
          fire_store(i * _NBUF + b, p, b)
      return carry

    lax.fori_loop(0, _NGROUP // 2, body, 0)

    for b in range(_NBUF):
      wait_store(1, b)

  return k


_kernel_call = _make_kernel()


def kernel(idx, weight):
  out_pad = _kernel_call(idx.astype(jnp.int32), weight)
  return out_pad[:, :, :_EMBED_DIM]
